# baseline (speedup 1.0000x reference)
import functools

import jax
import jax.numpy as jnp
from jax import lax
from jax.experimental import pallas as pl
from jax.experimental.pallas import tpu as pltpu
from jax.experimental.pallas import tpu_sc as plsc

N = 10000
E = 320000
D = 128
R = 64
B = 512

NC = 2
NS = 16
NW = NC * NS
K = 128
EPT = 10240
E_PAD = NW * EPT
NCHUNK = EPT // K
ROWS_PT = 640
N_ACC = NS * ROWS_PT

_MESH = plsc.VectorSubcoreMesh(core_axis_name="c", subcore_axis_name="s")


CF = N_ACC * R
CF_PT = CF // NS


NCH0 = 160
NCHC0 = 128
NCHC1 = 32
NCHT = NS * (NCHC0 + NCHC1)


def _sc_cnt_body(cidx_hbm, zr_hbm, cnt_out,
                 cidx2, onesv, acc_cnt, ssem):
    cid = lax.axis_index("c")
    sid = lax.axis_index("s")
    f0 = sid * CF_PT

    pltpu.sync_copy(zr_hbm.at[pl.ds(f0, CF_PT)], acc_cnt.at[pl.ds(f0, CF_PT)])
    ones16 = jnp.ones((16,), jnp.float32)
    for k in range(8):
        onesv[pl.ds(k * 16, 16)] = ones16
    c_base = jnp.where(cid == 0, sid * NCHC0, NS * NCHC0 + sid * NCHC1)

    def run(n_ch):
        pltpu.sync_copy(cidx_hbm.at[pl.ds(c_base, n_ch)],
                        cidx2.at[pl.ds(0, n_ch)])
        plsc.subcore_barrier()

        def fire(g, carry):
            descs = [pltpu.async_copy(onesv, acc_cnt.at[cidx2.at[g * MC + m]],
                                      ssem, add=True) for m in range(MC)]
            for dsc in descs:
                dsc.wait()
            return carry

        lax.fori_loop(0, n_ch // MC, fire, 0)

    @pl.when(cid == 0)
    def _():
        run(NCHC0)

    @pl.when(cid == 1)
    def _():
        run(NCHC1)

    plsc.subcore_barrier()
    pltpu.sync_copy(acc_cnt.at[pl.ds(f0, CF_PT)],
                    cnt_out.at[cid, pl.ds(f0, CF_PT)])


MC = 8
M = 8
NOUT = NCHUNK // M


def _sc_agg_body(x_hbm, pk_hbm, z_hbm, sx_out,
                 seg, srcv0, srcv1, dstv0, dstv1, rows0, rows1,
                 acc_sx, gs0, gs1, ss0, ss1):
    rows = [rows0, rows1]
    srcv = [srcv0, srcv1]
    dstv = [dstv0, dstv1]
    gsem = [gs0, gs1]
    ssem = [ss0, ss1]
    cid = lax.axis_index("c")
    sid = lax.axis_index("s")
    r0 = sid * ROWS_PT
    c_base = sid * NCH0

    def unpack(m, b):
        for k in range(8):
            sl = pl.ds(k * 16, 16)
            p = seg[m, sl]
            srcv[b][sl] = lax.bitwise_and(p, jnp.int32(0xFFFF))
            dstv[b][sl] = lax.shift_right_logical(p, jnp.int32(16))

    def scatter(b):
        return pltpu.async_copy(rows[b], acc_sx.at[dstv[b]], ssem[b], add=True)

    def body(g, carry):
        pltpu.sync_copy(pk_hbm.at[pl.ds(c_base + g * M, M)], seg)
        gd = [None] * M
        sd = [None] * M
        for m in range(M):
            b = m & 1
            if m >= 2:
                sd[m - 2].wait()
            unpack(m, b)
            gd[m] = pltpu.async_copy(x_hbm.at[srcv[b]], rows[b], gsem[b])
            if m >= 1:
                gd[m - 1].wait()
                sd[m - 1] = scatter((m - 1) & 1)
        gd[M - 1].wait()
        sd[M - 1] = scatter((M - 1) & 1)
        sd[M - 2].wait()
        sd[M - 1].wait()
        return carry

    @pl.when(cid == 0)
    def _():
        pltpu.sync_copy(z_hbm.at[pl.ds(r0, ROWS_PT)],
                        acc_sx.at[pl.ds(r0, ROWS_PT)])
        plsc.subcore_barrier()
        lax.fori_loop(0, NCH0 // M, body, 0)
        plsc.subcore_barrier()
        pltpu.sync_copy(acc_sx.at[pl.ds(r0, ROWS_PT)],
                        sx_out.at[pl.ds(r0, ROWS_PT)])


_sc_cnt = pl.kernel(
    _sc_cnt_body,
    out_type=jax.ShapeDtypeStruct((NC, CF), jnp.float32),
    mesh=_MESH,
    scratch_types=[
        pltpu.VMEM((NCH0, K), jnp.int32),
        pltpu.VMEM((K,), jnp.float32),
        pltpu.VMEM_SHARED((CF,), jnp.float32),
        pltpu.SemaphoreType.DMA,
    ],
)

_sc_agg = pl.kernel(
    _sc_agg_body,
    out_type=jax.ShapeDtypeStruct((N_ACC, D), jnp.float32),
    mesh=_MESH,
    scratch_types=(
        [pltpu.VMEM((M, K), jnp.int32)]
        + [pltpu.VMEM((K,), jnp.int32)] * 4
        + [pltpu.VMEM((K, D), jnp.float32)] * 2
        + [pltpu.VMEM_SHARED((N_ACC, D), jnp.float32)]
        + [pltpu.SemaphoreType.DMA] * 4
    ),
)


BLK = 1000
NBLK = N // BLK


def _tc1_body(sx_ref, cnt_ref, x_ref, rel_ref, wm_ref, wl_ref, wr_ref, b_ref,
              x1_ref, r1_ref, cnt_out_ref):
    sx = sx_ref[...]
    cnt = cnt_ref[0] + cnt_ref[1]
    cnt_out_ref[...] = cnt
    deg = jnp.sum(cnt, axis=1, keepdims=True)
    normv = 1.0 / jnp.maximum(deg, 1.0)
    sr = jnp.dot(cnt, rel_ref[...], preferred_element_type=jnp.float32)
    pre = (sx - sr) * normv
    h = (jnp.dot(pre, wm_ref[...], preferred_element_type=jnp.float32)
         + jnp.dot(x_ref[...], wl_ref[...], preferred_element_type=jnp.float32)
         + b_ref[...])
    x1_ref[...] = jnp.maximum(h, 0.0)

    @pl.when(pl.program_id(0) == 0)
    def _():
        r1_ref[...] = jnp.dot(rel_ref[...], wr_ref[...],
                              preferred_element_type=jnp.float32)


_tc1 = pl.pallas_call(
    _tc1_body,
    grid=(NBLK,),
    in_specs=[
        pl.BlockSpec((BLK, D), lambda i: (i, 0)),
        pl.BlockSpec((NC, BLK, R), lambda i: (0, i, 0)),
        pl.BlockSpec((BLK, D), lambda i: (i, 0)),
        pl.BlockSpec((R, D), lambda i: (0, 0)),
        pl.BlockSpec((D, D), lambda i: (0, 0)),
        pl.BlockSpec((D, D), lambda i: (0, 0)),
        pl.BlockSpec((D, D), lambda i: (0, 0)),
        pl.BlockSpec((1, D), lambda i: (0, 0)),
    ],
    out_specs=[
        pl.BlockSpec((BLK, D), lambda i: (i, 0)),
        pl.BlockSpec((R, D), lambda i: (0, 0)),
        pl.BlockSpec((BLK, R), lambda i: (i, 0)),
    ],
    out_shape=[
        jax.ShapeDtypeStruct((N, D), jnp.float32),
        jax.ShapeDtypeStruct((R, D), jnp.float32),
        jax.ShapeDtypeStruct((N, R), jnp.float32),
    ],
)


def _tc2_body(sx_ref, cnt_ref, x1_ref, r1_ref, bi_ref, wm_ref, wl_ref, b_ref,
              out_ref):
    i = pl.program_id(0)
    sx = sx_ref[...]
    cnt = cnt_ref[...]
    deg = jnp.sum(cnt, axis=1, keepdims=True)
    normv = 1.0 / jnp.maximum(deg, 1.0)
    sr = jnp.dot(cnt, r1_ref[...], preferred_element_type=jnp.float32)
    pre = (sx - sr) * normv
    x2 = (jnp.dot(pre, wm_ref[...], preferred_element_type=jnp.float32)
          + jnp.dot(x1_ref[...], wl_ref[...], preferred_element_type=jnp.float32)
          + b_ref[...])
    bi = bi_ref[0]
    sel = (lax.broadcasted_iota(jnp.int32, (B, BLK), 0) == bi).astype(jnp.float32)
    contrib = jnp.dot(sel, x2, preferred_element_type=jnp.float32)

    @pl.when(i == 0)
    def _():
        out_ref[...] = jnp.zeros_like(out_ref)

    out_ref[...] += contrib


_tc2 = pl.pallas_call(
    _tc2_body,
    grid=(NBLK,),
    in_specs=[
        pl.BlockSpec((BLK, D), lambda i: (i, 0)),
        pl.BlockSpec((BLK, R), lambda i: (i, 0)),
        pl.BlockSpec((BLK, D), lambda i: (i, 0)),
        pl.BlockSpec((R, D), lambda i: (0, 0)),
        pl.BlockSpec((1, 1, BLK), lambda i: (i, 0, 0)),
        pl.BlockSpec((D, D), lambda i: (0, 0)),
        pl.BlockSpec((D, D), lambda i: (0, 0)),
        pl.BlockSpec((1, D), lambda i: (0, 0)),
    ],
    out_specs=pl.BlockSpec((B, D), lambda i: (0, 0)),
    out_shape=jax.ShapeDtypeStruct((B, D), jnp.float32),
)


def kernel(ent_e, edge_index, edge_type, batch_idx, rel_e,
           W_msg1, W_loop1, W_rel1, b1,
           W_msg2, W_loop2, W_rel2, b2):
    pad = E_PAD - E
    src_p = jnp.concatenate([edge_index[0].astype(jnp.int32),
                             jnp.zeros((pad,), jnp.int32)])
    dst_p = jnp.concatenate([edge_index[1].astype(jnp.int32),
                             N + (jnp.arange(pad, dtype=jnp.int32) % (N_ACC - N))])
    et_p = jnp.concatenate([edge_type.astype(jnp.int32),
                            jnp.zeros((pad,), jnp.int32)])
    z = jnp.zeros((N_ACC, D), jnp.float32)
    zr = jnp.zeros((CF,), jnp.float32)

    pk2 = (dst_p * 65536 + src_p).reshape(NCHT, K)
    cidx2 = (dst_p * R + et_p).reshape(NCHT, K)

    cntp = _sc_cnt(cidx2, zr).reshape(NC, N_ACC, R)
    sxp = _sc_agg(ent_e, pk2, z)
    x1, r1, cnt = _tc1(sxp[:N], cntp[:, :N, :], ent_e, rel_e,
                       W_msg1, W_loop1, W_rel1, b1.reshape(1, D))
    sx2p = _sc_agg(x1, pk2, z)
    out = _tc2(sx2p[:N], cnt, x1, r1,
               batch_idx.reshape(NBLK, 1, BLK).astype(jnp.int32),
               W_msg2, W_loop2, b2.reshape(1, D))
    return out


# speedup vs baseline: 3.9247x; 3.9247x over previous
import jax
import jax.numpy as jnp
from jax import lax
from jax.experimental import pallas as pl
from jax.experimental.pallas import tpu as pltpu
from jax.experimental.pallas import tpu_sc as plsc

N = 10000
E = 320000
D = 128
R = 64
B = 512

NC = 2
NS = 16
NW = NC * NS
K = 128
NCH = 80
NCHT = NW * NCH
E_PAD = NCHT * K
ROWS_PT = 640
N_ACC = NS * ROWS_PT

MC = 8
M = 8

CF = N_ACC * R
CF_PT = CF // NS

_MESH = plsc.VectorSubcoreMesh(core_axis_name="c", subcore_axis_name="s")


def _sc_cnt_body(cidx_hbm, zr_hbm, cnt_out,
                 cidx2, onesv, acc_cnt, ssem):
    cid = lax.axis_index("c")
    sid = lax.axis_index("s")
    w = sid * NC + cid
    f0 = sid * CF_PT

    pltpu.sync_copy(zr_hbm.at[pl.ds(f0, CF_PT)], acc_cnt.at[pl.ds(f0, CF_PT)])
    pltpu.sync_copy(cidx_hbm.at[pl.ds(w * NCH, NCH)], cidx2)
    ones16 = jnp.ones((16,), jnp.float32)
    for k in range(8):
        onesv[pl.ds(k * 16, 16)] = ones16
    plsc.subcore_barrier()

    def fire(g, carry):
        descs = [pltpu.async_copy(onesv, acc_cnt.at[cidx2.at[g * MC + m]],
                                  ssem, add=True) for m in range(MC)]
        for dsc in descs:
            dsc.wait()
        return carry

    lax.fori_loop(0, NCH // MC, fire, 0)
    plsc.subcore_barrier()
    pltpu.sync_copy(acc_cnt.at[pl.ds(f0, CF_PT)],
                    cnt_out.at[cid, pl.ds(f0, CF_PT)])


def _sc_agg_body(x_hbm, pk_hbm, z_hbm, sx_out,
                 seg, srcv0, srcv1, dstv0, dstv1, rows0, rows1,
                 acc_sx, gs0, gs1, ss0, ss1):
    rows = [rows0, rows1]
    srcv = [srcv0, srcv1]
    dstv = [dstv0, dstv1]
    gsem = [gs0, gs1]
    ssem = [ss0, ss1]
    cid = lax.axis_index("c")
    sid = lax.axis_index("s")
    w = sid * NC + cid
    r0 = sid * ROWS_PT
    c_base = w * NCH

    pltpu.sync_copy(z_hbm.at[pl.ds(r0, ROWS_PT)], acc_sx.at[pl.ds(r0, ROWS_PT)])
    plsc.subcore_barrier()

    def unpack(m, b):
        for k in range(8):
            sl = pl.ds(k * 16, 16)
            p = seg[m, sl]
            srcv[b][sl] = lax.bitwise_and(p, jnp.int32(0xFFFF))
            dstv[b][sl] = lax.shift_right_logical(p, jnp.int32(16))

    def scatter(b):
        return pltpu.async_copy(rows[b], acc_sx.at[dstv[b]], ssem[b], add=True)

    def body(g, carry):
        pltpu.sync_copy(pk_hbm.at[pl.ds(c_base + g * M, M)], seg)
        gd = [None] * M
        sd = [None] * M
        for m in range(M):
            b = m & 1
            if m >= 2:
                sd[m - 2].wait()
            unpack(m, b)
            gd[m] = pltpu.async_copy(x_hbm.at[srcv[b]], rows[b], gsem[b])
            if m >= 1:
                gd[m - 1].wait()
                sd[m - 1] = scatter((m - 1) & 1)
        gd[M - 1].wait()
        sd[M - 1] = scatter((M - 1) & 1)
        sd[M - 2].wait()
        sd[M - 1].wait()
        return carry

    lax.fori_loop(0, NCH // M, body, 0)
    plsc.subcore_barrier()
    pltpu.sync_copy(acc_sx.at[pl.ds(r0, ROWS_PT)],
                    sx_out.at[cid, pl.ds(r0, ROWS_PT)])


_sc_cnt = pl.kernel(
    _sc_cnt_body,
    out_type=jax.ShapeDtypeStruct((NC, CF), jnp.float32),
    mesh=_MESH,
    scratch_types=[
        pltpu.VMEM((NCH, K), jnp.int32),
        pltpu.VMEM((K,), jnp.float32),
        pltpu.VMEM_SHARED((CF,), jnp.float32),
        pltpu.SemaphoreType.DMA,
    ],
)

_sc_agg = pl.kernel(
    _sc_agg_body,
    out_type=jax.ShapeDtypeStruct((NC, N_ACC, D), jnp.float32),
    mesh=_MESH,
    scratch_types=(
        [pltpu.VMEM((M, K), jnp.int32)]
        + [pltpu.VMEM((K,), jnp.int32)] * 4
        + [pltpu.VMEM((K, D), jnp.float32)] * 2
        + [pltpu.VMEM_SHARED((N_ACC, D), jnp.float32)]
        + [pltpu.SemaphoreType.DMA] * 4
    ),
)


BLK = 1000
NBLK = N // BLK


def _tc1_body(sx_ref, cnt_ref, x_ref, rel_ref, wm_ref, wl_ref, wr_ref, b_ref,
              x1_ref, r1_ref, cnt_out_ref):
    sx = sx_ref[0] + sx_ref[1]
    cnt = cnt_ref[0] + cnt_ref[1]
    cnt_out_ref[...] = cnt
    deg = jnp.sum(cnt, axis=1, keepdims=True)
    normv = 1.0 / jnp.maximum(deg, 1.0)
    sr = jnp.dot(cnt, rel_ref[...], preferred_element_type=jnp.float32)
    pre = (sx - sr) * normv
    h = (jnp.dot(pre, wm_ref[...], preferred_element_type=jnp.float32)
         + jnp.dot(x_ref[...], wl_ref[...], preferred_element_type=jnp.float32)
         + b_ref[...])
    x1_ref[...] = jnp.maximum(h, 0.0)

    @pl.when(pl.program_id(0) == 0)
    def _():
        r1_ref[...] = jnp.dot(rel_ref[...], wr_ref[...],
                              preferred_element_type=jnp.float32)


_tc1 = pl.pallas_call(
    _tc1_body,
    grid=(NBLK,),
    in_specs=[
        pl.BlockSpec((NC, BLK, D), lambda i: (0, i, 0)),
        pl.BlockSpec((NC, BLK, R), lambda i: (0, i, 0)),
        pl.BlockSpec((BLK, D), lambda i: (i, 0)),
        pl.BlockSpec((R, D), lambda i: (0, 0)),
        pl.BlockSpec((D, D), lambda i: (0, 0)),
        pl.BlockSpec((D, D), lambda i: (0, 0)),
        pl.BlockSpec((D, D), lambda i: (0, 0)),
        pl.BlockSpec((1, D), lambda i: (0, 0)),
    ],
    out_specs=[
        pl.BlockSpec((BLK, D), lambda i: (i, 0)),
        pl.BlockSpec((R, D), lambda i: (0, 0)),
        pl.BlockSpec((BLK, R), lambda i: (i, 0)),
    ],
    out_shape=[
        jax.ShapeDtypeStruct((N, D), jnp.float32),
        jax.ShapeDtypeStruct((R, D), jnp.float32),
        jax.ShapeDtypeStruct((N, R), jnp.float32),
    ],
)


def _tc2_body(sx_ref, cnt_ref, x1_ref, r1_ref, bi_ref, wm_ref, wl_ref, b_ref,
              out_ref):
    i = pl.program_id(0)
    sx = sx_ref[0] + sx_ref[1]
    cnt = cnt_ref[...]
    deg = jnp.sum(cnt, axis=1, keepdims=True)
    normv = 1.0 / jnp.maximum(deg, 1.0)
    sr = jnp.dot(cnt, r1_ref[...], preferred_element_type=jnp.float32)
    pre = (sx - sr) * normv
    x2 = (jnp.dot(pre, wm_ref[...], preferred_element_type=jnp.float32)
          + jnp.dot(x1_ref[...], wl_ref[...], preferred_element_type=jnp.float32)
          + b_ref[...])
    bi = bi_ref[0]
    sel = (lax.broadcasted_iota(jnp.int32, (B, BLK), 0) == bi).astype(jnp.float32)
    contrib = jnp.dot(sel, x2, preferred_element_type=jnp.float32)

    @pl.when(i == 0)
    def _():
        out_ref[...] = jnp.zeros_like(out_ref)

    out_ref[...] += contrib


_tc2 = pl.pallas_call(
    _tc2_body,
    grid=(NBLK,),
    in_specs=[
        pl.BlockSpec((NC, BLK, D), lambda i: (0, i, 0)),
        pl.BlockSpec((BLK, R), lambda i: (i, 0)),
        pl.BlockSpec((BLK, D), lambda i: (i, 0)),
        pl.BlockSpec((R, D), lambda i: (0, 0)),
        pl.BlockSpec((1, 1, BLK), lambda i: (i, 0, 0)),
        pl.BlockSpec((D, D), lambda i: (0, 0)),
        pl.BlockSpec((D, D), lambda i: (0, 0)),
        pl.BlockSpec((1, D), lambda i: (0, 0)),
    ],
    out_specs=pl.BlockSpec((B, D), lambda i: (0, 0)),
    out_shape=jax.ShapeDtypeStruct((B, D), jnp.float32),
)


def kernel(ent_e, edge_index, edge_type, batch_idx, rel_e,
           W_msg1, W_loop1, W_rel1, b1,
           W_msg2, W_loop2, W_rel2, b2):
    pad = E_PAD - E
    ar = jnp.arange(pad, dtype=jnp.int32)
    src_p = jnp.concatenate([edge_index[0].astype(jnp.int32), ar % N])
    dst_p = jnp.concatenate([edge_index[1].astype(jnp.int32),
                             N + (ar % (N_ACC - N))])
    et_p = jnp.concatenate([edge_type.astype(jnp.int32),
                            jnp.zeros((pad,), jnp.int32)])
    z = jnp.zeros((N_ACC, D), jnp.float32)
    zr = jnp.zeros((CF,), jnp.float32)

    def chunks(v):
        return (v.reshape(NCH, NW, K).transpose(1, 0, 2).reshape(NCHT, K))

    pk2 = chunks(dst_p * 65536 + src_p)
    cidx2 = chunks(dst_p * R + et_p)

    cntp = _sc_cnt(cidx2, zr).reshape(NC, N_ACC, R)
    sxp = _sc_agg(ent_e, pk2, z)
    x1, r1, cnt = _tc1(sxp[:, :N, :], cntp[:, :N, :], ent_e, rel_e,
                       W_msg1, W_loop1, W_rel1, b1.reshape(1, D))
    sx2p = _sc_agg(x1, pk2, z)
    out = _tc2(sx2p[:, :N, :], cnt, x1, r1,
               batch_idx.reshape(NBLK, 1, BLK).astype(jnp.int32),
               W_msg2, W_loop2, b2.reshape(1, D))
    return out


# speedup vs baseline: 4.1372x; 1.0542x over previous
import jax
import jax.numpy as jnp
from jax import lax
from jax.experimental import pallas as pl
from jax.experimental.pallas import tpu as pltpu
from jax.experimental.pallas import tpu_sc as plsc

N = 10000
E = 320000
D = 128
R = 64
B = 512

NC = 2
NS = 16
NW = NC * NS
K = 128
NCH = 80
NCHT = NW * NCH
E_PAD = NCHT * K
ROWS_PT = 640
N_ACC = NS * ROWS_PT

MC = 8
M = 16

CF = N_ACC * R
CF_PT = CF // NS

_MESH = plsc.VectorSubcoreMesh(core_axis_name="c", subcore_axis_name="s")


def _sc_cnt_body(cidx_hbm, zr_hbm, cnt_out,
                 cidx2, onesv, acc_cnt, ssem):
    cid = lax.axis_index("c")
    sid = lax.axis_index("s")
    w = sid * NC + cid
    f0 = sid * CF_PT

    pltpu.sync_copy(zr_hbm.at[pl.ds(f0, CF_PT)], acc_cnt.at[pl.ds(f0, CF_PT)])
    pltpu.sync_copy(cidx_hbm.at[pl.ds(w * NCH, NCH)], cidx2)
    ones16 = jnp.ones((16,), jnp.float32)
    for k in range(8):
        onesv[pl.ds(k * 16, 16)] = ones16
    plsc.subcore_barrier()

    def fire(g, carry):
        descs = [pltpu.async_copy(onesv, acc_cnt.at[cidx2.at[g * MC + m]],
                                  ssem, add=True) for m in range(MC)]
        for dsc in descs:
            dsc.wait()
        return carry

    lax.fori_loop(0, NCH // MC, fire, 0)
    plsc.subcore_barrier()
    pltpu.sync_copy(acc_cnt.at[pl.ds(f0, CF_PT)],
                    cnt_out.at[cid, pl.ds(f0, CF_PT)])


def _sc_agg_body(x_hbm, pk_hbm, z_hbm, sx_out,
                 seg, srcv0, srcv1, dstv0, dstv1, rows0, rows1,
                 acc_sx, gs0, gs1, ss0, ss1):
    rows = [rows0, rows1]
    srcv = [srcv0, srcv1]
    dstv = [dstv0, dstv1]
    gsem = [gs0, gs1]
    ssem = [ss0, ss1]
    cid = lax.axis_index("c")
    sid = lax.axis_index("s")
    w = sid * NC + cid
    r0 = sid * ROWS_PT
    c_base = w * NCH

    pltpu.sync_copy(z_hbm.at[pl.ds(r0, ROWS_PT)], acc_sx.at[pl.ds(r0, ROWS_PT)])
    plsc.subcore_barrier()

    def unpack(m, b):
        for k in range(8):
            sl = pl.ds(k * 16, 16)
            p = seg[m, sl]
            srcv[b][sl] = lax.bitwise_and(p, jnp.int32(0xFFFF))
            dstv[b][sl] = lax.shift_right_logical(p, jnp.int32(16))

    def scatter(b):
        return pltpu.async_copy(rows[b], acc_sx.at[dstv[b]], ssem[b], add=True)

    def body(g, carry):
        pltpu.sync_copy(pk_hbm.at[pl.ds(c_base + g * M, M)], seg)
        gd = [None] * M
        sd = [None] * M
        for m in range(M):
            b = m & 1
            if m >= 2:
                sd[m - 2].wait()
            unpack(m, b)
            gd[m] = pltpu.async_copy(x_hbm.at[srcv[b]], rows[b], gsem[b])
            if m >= 1:
                gd[m - 1].wait()
                sd[m - 1] = scatter((m - 1) & 1)
        gd[M - 1].wait()
        sd[M - 1] = scatter((M - 1) & 1)
        sd[M - 2].wait()
        sd[M - 1].wait()
        return carry

    lax.fori_loop(0, NCH // M, body, 0)
    plsc.subcore_barrier()
    pltpu.sync_copy(acc_sx.at[pl.ds(r0, ROWS_PT)],
                    sx_out.at[cid, pl.ds(r0, ROWS_PT)])


_sc_cnt = pl.kernel(
    _sc_cnt_body,
    out_type=jax.ShapeDtypeStruct((NC, CF), jnp.float32),
    mesh=_MESH,
    scratch_types=[
        pltpu.VMEM((NCH, K), jnp.int32),
        pltpu.VMEM((K,), jnp.float32),
        pltpu.VMEM_SHARED((CF,), jnp.float32),
        pltpu.SemaphoreType.DMA,
    ],
)

_sc_agg = pl.kernel(
    _sc_agg_body,
    out_type=jax.ShapeDtypeStruct((NC, N_ACC, D), jnp.float32),
    mesh=_MESH,
    scratch_types=(
        [pltpu.VMEM((M, K), jnp.int32)]
        + [pltpu.VMEM((K,), jnp.int32)] * 4
        + [pltpu.VMEM((K, D), jnp.float32)] * 2
        + [pltpu.VMEM_SHARED((N_ACC, D), jnp.float32)]
        + [pltpu.SemaphoreType.DMA] * 4
    ),
)


BLK = 1000
NBLK = N // BLK


def _tc1_body(sx_ref, cnt_ref, x_ref, rel_ref, wm_ref, wl_ref, wr_ref, b_ref,
              x1_ref, r1_ref, cnt_out_ref):
    sx = sx_ref[0] + sx_ref[1]
    cnt = cnt_ref[0] + cnt_ref[1]
    cnt_out_ref[...] = cnt
    deg = jnp.sum(cnt, axis=1, keepdims=True)
    normv = 1.0 / jnp.maximum(deg, 1.0)
    sr = jnp.dot(cnt, rel_ref[...], preferred_element_type=jnp.float32)
    pre = (sx - sr) * normv
    h = (jnp.dot(pre, wm_ref[...], preferred_element_type=jnp.float32)
         + jnp.dot(x_ref[...], wl_ref[...], preferred_element_type=jnp.float32)
         + b_ref[...])
    x1_ref[...] = jnp.maximum(h, 0.0)

    @pl.when(pl.program_id(0) == 0)
    def _():
        r1_ref[...] = jnp.dot(rel_ref[...], wr_ref[...],
                              preferred_element_type=jnp.float32)


_tc1 = pl.pallas_call(
    _tc1_body,
    grid=(NBLK,),
    in_specs=[
        pl.BlockSpec((NC, BLK, D), lambda i: (0, i, 0)),
        pl.BlockSpec((NC, BLK, R), lambda i: (0, i, 0)),
        pl.BlockSpec((BLK, D), lambda i: (i, 0)),
        pl.BlockSpec((R, D), lambda i: (0, 0)),
        pl.BlockSpec((D, D), lambda i: (0, 0)),
        pl.BlockSpec((D, D), lambda i: (0, 0)),
        pl.BlockSpec((D, D), lambda i: (0, 0)),
        pl.BlockSpec((1, D), lambda i: (0, 0)),
    ],
    out_specs=[
        pl.BlockSpec((BLK, D), lambda i: (i, 0)),
        pl.BlockSpec((R, D), lambda i: (0, 0)),
        pl.BlockSpec((BLK, R), lambda i: (i, 0)),
    ],
    out_shape=[
        jax.ShapeDtypeStruct((N, D), jnp.float32),
        jax.ShapeDtypeStruct((R, D), jnp.float32),
        jax.ShapeDtypeStruct((N, R), jnp.float32),
    ],
)


def _tc2_body(sx_ref, cnt_ref, x1_ref, r1_ref, bi_ref, wm_ref, wl_ref, b_ref,
              out_ref):
    i = pl.program_id(0)
    sx = sx_ref[0] + sx_ref[1]
    cnt = cnt_ref[...]
    deg = jnp.sum(cnt, axis=1, keepdims=True)
    normv = 1.0 / jnp.maximum(deg, 1.0)
    sr = jnp.dot(cnt, r1_ref[...], preferred_element_type=jnp.float32)
    pre = (sx - sr) * normv
    x2 = (jnp.dot(pre, wm_ref[...], preferred_element_type=jnp.float32)
          + jnp.dot(x1_ref[...], wl_ref[...], preferred_element_type=jnp.float32)
          + b_ref[...])
    bi = bi_ref[0]
    sel = (lax.broadcasted_iota(jnp.int32, (B, BLK), 0) == bi).astype(jnp.float32)
    contrib = jnp.dot(sel, x2, preferred_element_type=jnp.float32)

    @pl.when(i == 0)
    def _():
        out_ref[...] = jnp.zeros_like(out_ref)

    out_ref[...] += contrib


_tc2 = pl.pallas_call(
    _tc2_body,
    grid=(NBLK,),
    in_specs=[
        pl.BlockSpec((NC, BLK, D), lambda i: (0, i, 0)),
        pl.BlockSpec((BLK, R), lambda i: (i, 0)),
        pl.BlockSpec((BLK, D), lambda i: (i, 0)),
        pl.BlockSpec((R, D), lambda i: (0, 0)),
        pl.BlockSpec((1, 1, BLK), lambda i: (i, 0, 0)),
        pl.BlockSpec((D, D), lambda i: (0, 0)),
        pl.BlockSpec((D, D), lambda i: (0, 0)),
        pl.BlockSpec((1, D), lambda i: (0, 0)),
    ],
    out_specs=pl.BlockSpec((B, D), lambda i: (0, 0)),
    out_shape=jax.ShapeDtypeStruct((B, D), jnp.float32),
)


def kernel(ent_e, edge_index, edge_type, batch_idx, rel_e,
           W_msg1, W_loop1, W_rel1, b1,
           W_msg2, W_loop2, W_rel2, b2):
    pad = E_PAD - E
    ar = jnp.arange(pad, dtype=jnp.int32)
    src_p = jnp.concatenate([edge_index[0].astype(jnp.int32), ar % N])
    dst_p = jnp.concatenate([edge_index[1].astype(jnp.int32),
                             N + (ar % (N_ACC - N))])
    et_p = jnp.concatenate([edge_type.astype(jnp.int32),
                            jnp.zeros((pad,), jnp.int32)])
    z = jnp.zeros((N_ACC, D), jnp.float32)
    zr = jnp.zeros((CF,), jnp.float32)

    def chunks(v):
        return (v.reshape(NCH, NW, K).transpose(1, 0, 2).reshape(NCHT, K))

    pk2 = chunks(dst_p * 65536 + src_p)
    cidx2 = chunks(dst_p * R + et_p)

    cntp = _sc_cnt(cidx2, zr).reshape(NC, N_ACC, R)
    sxp = _sc_agg(ent_e, pk2, z)
    x1, r1, cnt = _tc1(sxp[:, :N, :], cntp[:, :N, :], ent_e, rel_e,
                       W_msg1, W_loop1, W_rel1, b1.reshape(1, D))
    sx2p = _sc_agg(x1, pk2, z)
    out = _tc2(sx2p[:, :N, :], cnt, x1, r1,
               batch_idx.reshape(NBLK, 1, BLK).astype(jnp.int32),
               W_msg2, W_loop2, b2.reshape(1, D))
    return out
